# baseline (speedup 1.0000x reference)
import functools

import jax
import jax.numpy as jnp
from jax import lax
from jax.experimental import pallas as pl
from jax.experimental.pallas import tpu as pltpu
from jax.experimental.pallas import tpu_sc as plsc

B = 16384
D = 128
H = 1024

_INFO = plsc.get_sparse_core_info()
_NC, _NS = _INFO.num_cores, _INFO.num_subcores
_NW = _NC * _NS
_BPW = B // _NW
_CH = 128
_NCHUNK = _BPW // _CH

_sc_mesh = plsc.VectorSubcoreMesh(core_axis_name="c", subcore_axis_name="s")


@functools.partial(
    pl.kernel,
    mesh=_sc_mesh,
    out_type=[
        jax.ShapeDtypeStruct((B, D), jnp.float32),
        jax.ShapeDtypeStruct((B, D), jnp.float32),
    ],
    scratch_types=[
        pltpu.VMEM((_NCHUNK, _CH), jnp.int32),
        pltpu.VMEM((_BPW, D), jnp.float32),
        pltpu.SemaphoreType.DMA,
    ],
)
def _sc_gather(xu_hbm, xm_hbm, ut_hbm, mt_hbm, u_out, m_out, idx_v, rows_v, sem):
    wid = lax.axis_index("s") * _NC + lax.axis_index("c")
    base = wid * _BPW
    cbase = wid * _NCHUNK

    pltpu.sync_copy(xu_hbm.at[pl.ds(cbase, _NCHUNK)], idx_v)
    cps = [
        pltpu.async_copy(
            ut_hbm.at[idx_v.at[j]], rows_v.at[pl.ds(j * _CH, _CH)], sem
        )
        for j in range(_NCHUNK)
    ]
    for cp in cps:
        cp.wait()
    pltpu.sync_copy(rows_v, u_out.at[pl.ds(base, _BPW)])

    pltpu.sync_copy(xm_hbm.at[pl.ds(cbase, _NCHUNK)], idx_v)
    cps = [
        pltpu.async_copy(
            mt_hbm.at[idx_v.at[j]], rows_v.at[pl.ds(j * _CH, _CH)], sem
        )
        for j in range(_NCHUNK)
    ]
    for cp in cps:
        cp.wait()
    pltpu.sync_copy(rows_v, m_out.at[pl.ds(base, _BPW)])


_BB = 2048


def _mlp_body(u_ref, m_ref, w1u_ref, w1m_ref, b1_ref, w2_ref, b2_ref, out_ref):
    ub = u_ref[...].astype(jnp.bfloat16)
    mb = m_ref[...].astype(jnp.bfloat16)
    h = (
        jnp.dot(ub, w1u_ref[...], preferred_element_type=jnp.float32)
        + jnp.dot(mb, w1m_ref[...], preferred_element_type=jnp.float32)
        + b1_ref[...]
    )
    h = jnp.maximum(h, 0.0)
    out_ref[...] = jnp.sum(h * w2_ref[...], axis=1, keepdims=True) + b2_ref[0, 0]


def _mlp(u, m, w1u, w1m, b1r, w2r, b2r):
    return pl.pallas_call(
        _mlp_body,
        grid=(B // _BB,),
        in_specs=[
            pl.BlockSpec((_BB, D), lambda i: (i, 0)),
            pl.BlockSpec((_BB, D), lambda i: (i, 0)),
            pl.BlockSpec((D, H), lambda i: (0, 0)),
            pl.BlockSpec((D, H), lambda i: (0, 0)),
            pl.BlockSpec((1, H), lambda i: (0, 0)),
            pl.BlockSpec((1, H), lambda i: (0, 0)),
            pl.BlockSpec((1, 1), lambda i: (0, 0)),
        ],
        out_specs=pl.BlockSpec((_BB, 1), lambda i: (i, 0)),
        out_shape=jax.ShapeDtypeStruct((B, 1), jnp.float32),
    )(u, m, w1u, w1m, b1r, w2r, b2r)


def kernel(X, user_table, movie_table, W1, b1, W2, b2):
    xu = X[:, 0].astype(jnp.int32).reshape(B // _CH, _CH)
    xm = X[:, 1].astype(jnp.int32).reshape(B // _CH, _CH)
    u_embed, m_embed = _sc_gather(xu, xm, user_table, movie_table)
    w1u = W1[:D].astype(jnp.bfloat16)
    w1m = W1[D:].astype(jnp.bfloat16)
    b1r = b1.reshape(1, H)
    w2r = W2.reshape(1, H)
    b2r = b2.reshape(1, 1)
    return _mlp(u_embed, m_embed, w1u, w1m, b1r, w2r, b2r)


# speedup vs baseline: 1.0174x; 1.0174x over previous
import functools

import jax
import jax.numpy as jnp
from jax import lax
from jax.experimental import pallas as pl
from jax.experimental.pallas import tpu as pltpu
from jax.experimental.pallas import tpu_sc as plsc

B = 16384
D = 128
H = 1024

_INFO = plsc.get_sparse_core_info()
_NC, _NS = _INFO.num_cores, _INFO.num_subcores
_NW = _NC * _NS
_BPW = B // _NW
_CH = 128
_NCHUNK = _BPW // _CH

_sc_mesh = plsc.VectorSubcoreMesh(core_axis_name="c", subcore_axis_name="s")


@functools.partial(
    pl.kernel,
    mesh=_sc_mesh,
    out_type=[
        jax.ShapeDtypeStruct((B, D), jnp.float32),
        jax.ShapeDtypeStruct((B, D), jnp.float32),
    ],
    scratch_types=[
        pltpu.VMEM((_NCHUNK, _CH), jnp.int32),
        pltpu.VMEM((_BPW, D), jnp.float32),
        pltpu.SemaphoreType.DMA,
    ],
)
def _sc_gather(xu_hbm, xm_hbm, ut_hbm, mt_hbm, u_out, m_out, idx_v, rows_v, sem):
    wid = lax.axis_index("s") * _NC + lax.axis_index("c")
    base = wid * _BPW
    cbase = wid * _NCHUNK

    pltpu.sync_copy(xu_hbm.at[pl.ds(cbase, _NCHUNK)], idx_v)
    cps = [
        pltpu.async_copy(
            ut_hbm.at[idx_v.at[j]], rows_v.at[pl.ds(j * _CH, _CH)], sem
        )
        for j in range(_NCHUNK)
    ]
    for cp in cps:
        cp.wait()
    pltpu.sync_copy(rows_v, u_out.at[pl.ds(base, _BPW)])

    pltpu.sync_copy(xm_hbm.at[pl.ds(cbase, _NCHUNK)], idx_v)
    cps = [
        pltpu.async_copy(
            mt_hbm.at[idx_v.at[j]], rows_v.at[pl.ds(j * _CH, _CH)], sem
        )
        for j in range(_NCHUNK)
    ]
    for cp in cps:
        cp.wait()
    pltpu.sync_copy(rows_v, m_out.at[pl.ds(base, _BPW)])


_BB = 2048


def _mlp_body(u_ref, m_ref, w1u_ref, w1m_ref, b1_ref, w2_ref, b2_ref, out_ref):
    ub = u_ref[...].astype(jnp.bfloat16)
    mb = m_ref[...].astype(jnp.bfloat16)
    h = (
        jnp.dot(ub, w1u_ref[...], preferred_element_type=jnp.float32)
        + jnp.dot(mb, w1m_ref[...], preferred_element_type=jnp.float32)
        + b1_ref[...]
    )
    h = jnp.maximum(h, 0.0)
    g = lax.dot_general(
        w2_ref[...], h, (((1,), (1,)), ((), ())),
        preferred_element_type=jnp.float32,
    )
    out_ref[...] = g.reshape(_BB // 128, 128) + b2_ref[0, 0]


def _mlp(u, m, w1u, w1m, b1r, w2r, b2r):
    return pl.pallas_call(
        _mlp_body,
        grid=(B // _BB,),
        in_specs=[
            pl.BlockSpec((_BB, D), lambda i: (i, 0)),
            pl.BlockSpec((_BB, D), lambda i: (i, 0)),
            pl.BlockSpec((D, H), lambda i: (0, 0)),
            pl.BlockSpec((D, H), lambda i: (0, 0)),
            pl.BlockSpec((1, H), lambda i: (0, 0)),
            pl.BlockSpec((1, H), lambda i: (0, 0)),
            pl.BlockSpec((1, 1), lambda i: (0, 0)),
        ],
        out_specs=pl.BlockSpec((_BB // 128, 128), lambda i: (i, 0)),
        out_shape=jax.ShapeDtypeStruct((B // 128, 128), jnp.float32),
    )(u, m, w1u, w1m, b1r, w2r, b2r)


def kernel(X, user_table, movie_table, W1, b1, W2, b2):
    xu = X[:, 0].astype(jnp.int32).reshape(B // _CH, _CH)
    xm = X[:, 1].astype(jnp.int32).reshape(B // _CH, _CH)
    u_embed, m_embed = _sc_gather(xu, xm, user_table, movie_table)
    w1u = W1[:D].astype(jnp.bfloat16)
    w1m = W1[D:].astype(jnp.bfloat16)
    b1r = b1.reshape(1, H)
    w2r = W2.reshape(1, H)
    b2r = b2.reshape(1, 1)
    out = _mlp(u_embed, m_embed, w1u, w1m, b1r, w2r, b2r)
    return out.reshape(B, 1)


# speedup vs baseline: 1.0725x; 1.0541x over previous
import functools

import jax
import jax.numpy as jnp
from jax import lax
from jax.experimental import pallas as pl
from jax.experimental.pallas import tpu as pltpu
from jax.experimental.pallas import tpu_sc as plsc

B = 16384
D = 128
H = 1024

_INFO = plsc.get_sparse_core_info()
_NC, _NS = _INFO.num_cores, _INFO.num_subcores
_NW = _NC * _NS
_BPW = B // _NW
_CH = 128
_NCHUNK = _BPW // _CH

_sc_mesh = plsc.VectorSubcoreMesh(core_axis_name="c", subcore_axis_name="s")


@functools.partial(
    pl.kernel,
    mesh=_sc_mesh,
    out_type=[
        jax.ShapeDtypeStruct((B, D), jnp.float32),
        jax.ShapeDtypeStruct((B, D), jnp.float32),
    ],
    scratch_types=[
        pltpu.VMEM((_NCHUNK, _CH), jnp.int32),
        pltpu.VMEM((_BPW, D), jnp.float32),
        pltpu.SemaphoreType.DMA,
    ],
)
def _sc_gather(xu_hbm, xm_hbm, ut_hbm, mt_hbm, u_out, m_out, idx_v, rows_v, sem):
    wid = lax.axis_index("s") * _NC + lax.axis_index("c")
    base = wid * _BPW
    cbase = wid * _NCHUNK

    pltpu.sync_copy(xu_hbm.at[pl.ds(cbase, _NCHUNK)], idx_v)
    cps = [
        pltpu.async_copy(
            ut_hbm.at[idx_v.at[j]], rows_v.at[pl.ds(j * _CH, _CH)], sem
        )
        for j in range(_NCHUNK)
    ]
    for cp in cps:
        cp.wait()
    pltpu.sync_copy(rows_v, u_out.at[pl.ds(base, _BPW)])

    pltpu.sync_copy(xm_hbm.at[pl.ds(cbase, _NCHUNK)], idx_v)
    cps = [
        pltpu.async_copy(
            mt_hbm.at[idx_v.at[j]], rows_v.at[pl.ds(j * _CH, _CH)], sem
        )
        for j in range(_NCHUNK)
    ]
    for cp in cps:
        cp.wait()
    pltpu.sync_copy(rows_v, m_out.at[pl.ds(base, _BPW)])


_BB = 2048


def _mlp_body(u_ref, m_ref, w1ut_ref, w1mt_ref, b1_ref, w2_ref, b2_ref, out_ref):
    ut = u_ref[...].astype(jnp.bfloat16).T
    mt = m_ref[...].astype(jnp.bfloat16).T
    hT = (
        jnp.dot(w1ut_ref[...], ut, preferred_element_type=jnp.float32)
        + jnp.dot(w1mt_ref[...], mt, preferred_element_type=jnp.float32)
        + b1_ref[...]
    )
    hT = jnp.maximum(hT, 0.0).astype(jnp.bfloat16)
    g = jnp.dot(w2_ref[...], hT, preferred_element_type=jnp.float32)
    out_ref[...] = g.reshape(_BB // 128, 128) + b2_ref[0, 0]


def _mlp(u, m, w1u, w1m, b1r, w2r, b2r):
    return pl.pallas_call(
        _mlp_body,
        grid=(B // _BB,),
        in_specs=[
            pl.BlockSpec((_BB, D), lambda i: (i, 0)),
            pl.BlockSpec((_BB, D), lambda i: (i, 0)),
            pl.BlockSpec((H, D), lambda i: (0, 0)),
            pl.BlockSpec((H, D), lambda i: (0, 0)),
            pl.BlockSpec((H, 1), lambda i: (0, 0)),
            pl.BlockSpec((1, H), lambda i: (0, 0)),
            pl.BlockSpec((1, 1), lambda i: (0, 0)),
        ],
        out_specs=pl.BlockSpec((_BB // 128, 128), lambda i: (i, 0)),
        out_shape=jax.ShapeDtypeStruct((B // 128, 128), jnp.float32),
    )(u, m, w1u, w1m, b1r, w2r, b2r)


def kernel(X, user_table, movie_table, W1, b1, W2, b2):
    xu = X[:, 0].astype(jnp.int32).reshape(B // _CH, _CH)
    xm = X[:, 1].astype(jnp.int32).reshape(B // _CH, _CH)
    u_embed, m_embed = _sc_gather(xu, xm, user_table, movie_table)
    w1ut = W1[:D].T.astype(jnp.bfloat16)
    w1mt = W1[D:].T.astype(jnp.bfloat16)
    b1r = b1.reshape(H, 1)
    w2r = W2.reshape(1, H).astype(jnp.bfloat16)
    b2r = b2.reshape(1, 1)
    out = _mlp(u_embed, m_embed, w1ut, w1mt, b1r, w2r, b2r)
    return out.reshape(B, 1)
